# baseline (speedup 1.0000x reference)
import jax
import jax.numpy as jnp
from jax import lax
from jax.experimental import pallas as pl
from jax.experimental.pallas import tpu as pltpu
from jax.experimental.pallas import tpu_sc as plsc

N_TRACKS, Q, N = 256, 8, 128
B = 4096
EPS = 1e-09
NC, NS = 2, 16
NW = NC * NS
TPW = N_TRACKS // NW
RPW = TPW * Q
CHUNK = 128
NCH = B // CHUNK
NK = N // 16


def _sc_update(reprs_hbm, tids_hbm, mem_hbm, alpha_hbm, out_hbm,
               tid_v, bb2_v, tb1_v, rows_v, mem_v, chos_v, alp_v, out_v,
               minv_s, sem):
    cid = lax.axis_index("c")
    sid = lax.axis_index("s")
    wid = sid * NC + cid
    lo = wid * TPW

    pltpu.sync_copy(tids_hbm, tid_v)
    pltpu.sync_copy(mem_hbm.at[pl.ds(lo, TPW)], mem_v)
    pltpu.sync_copy(alpha_hbm, alp_v)

    zero16 = jnp.zeros((16,), jnp.int32)
    for row in range(NCH + 1):
        for i in range(CHUNK // 16):
            bb2_v.at[row][pl.ds(i * 16, 16)] = zero16

    def _init(i, c):
        minv_s[i] = jnp.float32(jnp.inf)
        return c
    lax.fori_loop(0, RPW, _init, 0)

    lane_iota = lax.iota(jnp.int32, 16)
    def _bin(i, cur):
        v = tid_v[pl.ds(i * 16, 16)]
        m = (v >= lo) & (v < lo + TPW)
        cs = plsc.cumsum(jnp.where(m, jnp.int32(1), jnp.int32(0)))
        pos = jnp.where(m, cur + cs - 1, B + lane_iota)
        plsc.store_scatter(bb2_v, [pos // CHUNK, pos % CHUNK],
                           lane_iota + i * 16)
        plsc.store_scatter(tb1_v, [pos], v)
        return cur + cs[15]
    nb = lax.fori_loop(0, B // 16, _bin, 0, unroll=4)

    def _chunk(c, carry):
        base = c * CHUNK
        pltpu.async_copy(reprs_hbm.at[bb2_v.at[c]], rows_v, sem).wait()
        nj = jnp.minimum(CHUNK, nb - base)

        def _b(j, c2):
            tl = jnp.clip(tb1_v[pl.ds(base + j, 16)][0] - lo, 0, TPW - 1)
            rr = rows_v.at[j]
            r = [rr[pl.ds(k * 16, 16)] for k in range(NK)]
            for q in range(Q):
                mr = mem_v.at[tl, q]
                acc = r[0] * mr[pl.ds(0, 16)]
                for k in range(1, NK):
                    acc = acc + r[k] * mr[pl.ds(k * 16, 16)]
                s = jnp.sum(acc)
                sl = tl * Q + q
                cu = minv_s[sl]
                upd = s < cu
                minv_s[sl] = jnp.where(upd, s, cu)

                @pl.when(upd)
                def _copy_row(sl=sl, r=r):
                    cc = chos_v.at[sl]
                    for k in range(NK):
                        cc[pl.ds(k * 16, 16)] = r[k]
            return c2
        lax.fori_loop(0, nj, _b, 0)
        return carry
    nchunks = (nb + CHUNK - 1) // CHUNK
    lax.fori_loop(0, nchunks, _chunk, 0)

    half = jnp.full((16,), 0.5, jnp.float32)
    three_half = jnp.full((16,), 1.5, jnp.float32)
    eps = jnp.full((16,), EPS, jnp.float32)
    magic = jnp.full((16,), 0x5f3759df, jnp.int32)
    for q in range(Q):
        aq = alp_v.at[q]
        av = [aq[pl.ds(k * 16, 16)] for k in range(NK)]
        for t in range(TPW):
            sl = t * Q + q
            present = minv_s[sl] < jnp.float32(jnp.inf)
            ov = out_v.at[sl]
            mr = mem_v.at[t, q]

            @pl.when(present)
            def _blend(mr=mr, ov=ov, sl=sl, av=av):
                cc = chos_v.at[sl]
                nk = []
                ssq = None
                for k in range(NK):
                    mk = mr[pl.ds(k * 16, 16)]
                    ck = cc[pl.ds(k * 16, 16)]
                    nv = ck + av[k] * (mk - ck)
                    nk.append(nv)
                    ssq = nv * nv if ssq is None else ssq + nv * nv
                cs = plsc.cumsum(ssq)
                rcs = lax.rev(plsc.cumsum(lax.rev(ssq, (0,))), (0,))
                tot = cs + rcs - ssq
                u = plsc.bitcast(tot, jnp.int32)
                y = plsc.bitcast(magic - lax.shift_right_logical(u, 1),
                                 jnp.float32)
                for _ in range(3):
                    y = y * (three_half - ((half * tot) * y) * y)
                den = tot * y + eps
                for k in range(NK):
                    ov[pl.ds(k * 16, 16)] = nk[k] / den

            @pl.when(jnp.logical_not(present))
            def _passthru(mr=mr, ov=ov):
                for k in range(NK):
                    ov[pl.ds(k * 16, 16)] = mr[pl.ds(k * 16, 16)]

    pltpu.sync_copy(out_v, out_hbm.at[pl.ds(lo * Q, RPW)])


_sc_call = pl.kernel(
    _sc_update,
    out_type=jax.ShapeDtypeStruct((N_TRACKS * Q, N), jnp.float32),
    mesh=plsc.VectorSubcoreMesh(core_axis_name="c", subcore_axis_name="s",
                                num_cores=NC, num_subcores=NS),
    compiler_params=pltpu.CompilerParams(needs_layout_passes=False),
    scratch_types=[
        pltpu.VMEM((B,), jnp.int32),
        pltpu.VMEM((NCH + 1, CHUNK), jnp.int32),
        pltpu.VMEM((B + CHUNK,), jnp.int32),
        pltpu.VMEM((CHUNK, N), jnp.float32),
        pltpu.VMEM((TPW, Q, N), jnp.float32),
        pltpu.VMEM((RPW, N), jnp.float32),
        pltpu.VMEM((Q, N), jnp.float32),
        pltpu.VMEM((RPW, N), jnp.float32),
        pltpu.SMEM((RPW,), jnp.float32),
        pltpu.SemaphoreType.DMA,
    ],
)


@jax.jit
def kernel(reprs, track_idxs, memory, alpha):
    tids = track_idxs.astype(jnp.int32)
    alpha_b = jnp.broadcast_to(alpha.reshape(Q, 1), (Q, N))
    out = _sc_call(reprs, tids, memory, alpha_b)
    return out.reshape(N_TRACKS, Q, N)


# speedup vs baseline: 1.0672x; 1.0672x over previous
import jax
import jax.numpy as jnp
from jax import lax
from jax.experimental import pallas as pl
from jax.experimental.pallas import tpu as pltpu
from jax.experimental.pallas import tpu_sc as plsc

N_TRACKS, Q, N = 256, 8, 128
B = 4096
EPS = 1e-09
NC, NS = 2, 16
NW = NC * NS
TPW = N_TRACKS // NW
RPW = TPW * Q
CHUNK = 128
NCH = B // CHUNK
NK = N // 16


def _sc_update(reprs_hbm, tids_hbm, mem_hbm, chosen_hbm,
               tid_v, bb2_v, tb1_v, rows_v, mem_v, chos_v, minv_s, sem):
    cid = lax.axis_index("c")
    sid = lax.axis_index("s")
    wid = sid * NC + cid
    lo = wid * TPW

    pltpu.sync_copy(tids_hbm, tid_v)
    pltpu.sync_copy(mem_hbm.at[pl.ds(lo, TPW)], mem_v)

    for t in range(TPW):
        for q in range(Q):
            src = mem_v.at[t, q]
            dst = chos_v.at[t * Q + q]
            for k in range(NK):
                dst[pl.ds(k * 16, 16)] = src[pl.ds(k * 16, 16)]

    zero16 = jnp.zeros((16,), jnp.int32)
    for row in range(NCH + 1):
        for i in range(CHUNK // 16):
            bb2_v.at[row][pl.ds(i * 16, 16)] = zero16

    def _init(i, c):
        minv_s[i] = jnp.float32(jnp.inf)
        return c
    lax.fori_loop(0, RPW, _init, 0)

    lane_iota = lax.iota(jnp.int32, 16)
    def _bin(i, cur):
        v = tid_v[pl.ds(i * 16, 16)]
        m = (v >= lo) & (v < lo + TPW)
        cs = plsc.cumsum(jnp.where(m, jnp.int32(1), jnp.int32(0)))
        pos = jnp.where(m, cur + cs - 1, B + lane_iota)
        plsc.store_scatter(bb2_v, [pos // CHUNK, pos % CHUNK],
                           lane_iota + i * 16)
        plsc.store_scatter(tb1_v, [pos], v)
        return cur + cs[15]
    nb = lax.fori_loop(0, B // 16, _bin, 0, unroll=4)

    def _chunk(c, carry):
        base = c * CHUNK
        pltpu.async_copy(reprs_hbm.at[bb2_v.at[c]], rows_v, sem).wait()
        nj = jnp.minimum(CHUNK, nb - base)

        def _b(j, c2):
            tl = jnp.clip(tb1_v[pl.ds(base + j, 16)][0] - lo, 0, TPW - 1)
            rr = rows_v.at[j]
            r = [rr[pl.ds(k * 16, 16)] for k in range(NK)]
            for q in range(Q):
                mr = mem_v.at[tl, q]
                acc = r[0] * mr[pl.ds(0, 16)]
                for k in range(1, NK):
                    acc = acc + r[k] * mr[pl.ds(k * 16, 16)]
                s = jnp.sum(acc)
                sl = tl * Q + q
                cu = minv_s[sl]
                upd = s < cu
                minv_s[sl] = jnp.where(upd, s, cu)

                @pl.when(upd)
                def _copy_row(sl=sl, r=r):
                    cc = chos_v.at[sl]
                    for k in range(NK):
                        cc[pl.ds(k * 16, 16)] = r[k]
            return c2
        lax.fori_loop(0, nj, _b, 0)
        return carry
    nchunks = (nb + CHUNK - 1) // CHUNK
    lax.fori_loop(0, nchunks, _chunk, 0)

    pltpu.sync_copy(chos_v, chosen_hbm.at[pl.ds(lo * Q, RPW)])


_sc_call = pl.kernel(
    _sc_update,
    out_type=jax.ShapeDtypeStruct((N_TRACKS * Q, N), jnp.float32),
    mesh=plsc.VectorSubcoreMesh(core_axis_name="c", subcore_axis_name="s",
                                num_cores=NC, num_subcores=NS),
    compiler_params=pltpu.CompilerParams(needs_layout_passes=False),
    scratch_types=[
        pltpu.VMEM((B,), jnp.int32),
        pltpu.VMEM((NCH + 1, CHUNK), jnp.int32),
        pltpu.VMEM((B + CHUNK,), jnp.int32),
        pltpu.VMEM((CHUNK, N), jnp.float32),
        pltpu.VMEM((TPW, Q, N), jnp.float32),
        pltpu.VMEM((RPW, N), jnp.float32),
        pltpu.SMEM((RPW,), jnp.float32),
        pltpu.SemaphoreType.DMA,
    ],
)


def _finish_kernel(mem_ref, chosen_ref, tids_ref, alpha_ref, out_ref):
    mem = mem_ref[...]
    ch = chosen_ref[...].reshape(N_TRACKS, Q, N)
    a = alpha_ref[...].reshape(1, Q, N)
    new = mem * a + ch * (1.0 - a)
    nrm = jnp.sqrt(jnp.sum(new * new, axis=-1, keepdims=True))
    new = new / (nrm + EPS)
    row_t = jax.lax.broadcasted_iota(jnp.int32, (N_TRACKS, B), 0)
    present = jnp.any(row_t == tids_ref[...].reshape(1, B), axis=1)
    out_ref[...] = jnp.where(present.reshape(N_TRACKS, 1, 1), new, mem)


@jax.jit
def kernel(reprs, track_idxs, memory, alpha):
    tids = track_idxs.astype(jnp.int32)
    chosen = _sc_call(reprs, tids, memory)
    alpha_b = jnp.broadcast_to(alpha.reshape(Q, 1), (Q, N))
    out = pl.pallas_call(
        _finish_kernel,
        out_shape=jax.ShapeDtypeStruct((N_TRACKS, Q, N), jnp.float32),
    )(memory, chosen, tids.reshape(1, B), alpha_b)
    return out


# speedup vs baseline: 1.4174x; 1.3282x over previous
import jax
import jax.numpy as jnp
from jax import lax
from jax.experimental import pallas as pl
from jax.experimental.pallas import tpu as pltpu
from jax.experimental.pallas import tpu_sc as plsc

N_TRACKS, Q, N = 256, 8, 128
B = 4096
EPS = 1e-09
NC, NS = 2, 16
NW = NC * NS
TPW = N_TRACKS // NW
RPW = TPW * Q
CHUNK = 128
NCH = B // CHUNK
NK = N // 16


def _sc_update(reprs_hbm, tids_hbm, mem_hbm, chosen_hbm,
               tid_v, bb1_v, bb2_v, tb1_v, rows_v, mem_v, chos_v, idx_v,
               minv_s, minb_s, sem):
    cid = lax.axis_index("c")
    sid = lax.axis_index("s")
    wid = sid * NC + cid
    lo = wid * TPW

    pltpu.sync_copy(tids_hbm, tid_v)
    pltpu.sync_copy(mem_hbm.at[pl.ds(lo, TPW)], mem_v)

    zero16 = jnp.zeros((16,), jnp.int32)
    for row in range(NCH + 1):
        for i in range(CHUNK // 16):
            bb2_v.at[row][pl.ds(i * 16, 16)] = zero16

    def _init(i, c):
        minv_s[i] = jnp.float32(jnp.inf)
        minb_s[i] = -1
        return c
    lax.fori_loop(0, RPW, _init, 0)

    lane_iota = lax.iota(jnp.int32, 16)
    def _bin(i, cur):
        v = tid_v[pl.ds(i * 16, 16)]
        m = (v >= lo) & (v < lo + TPW)
        cs = plsc.cumsum(jnp.where(m, jnp.int32(1), jnp.int32(0)))
        pos = jnp.where(m, cur + cs - 1, B + lane_iota)
        bi = lane_iota + i * 16
        plsc.store_scatter(bb2_v, [pos // CHUNK, pos % CHUNK], bi)
        plsc.store_scatter(bb1_v, [pos], bi)
        plsc.store_scatter(tb1_v, [pos], v)
        return cur + cs[15]
    nb = lax.fori_loop(0, B // 16, _bin, 0, unroll=4)

    def _chunk(c, carry):
        base = c * CHUNK
        pltpu.async_copy(reprs_hbm.at[bb2_v.at[c]], rows_v, sem).wait()
        nj = jnp.minimum(CHUNK, nb - base)

        def _b(j, c2):
            tl = jnp.clip(tb1_v[pl.ds(base + j, 16)][0] - lo, 0, TPW - 1)
            bg = bb1_v[pl.ds(base + j, 16)][0]
            rr = rows_v.at[j]
            r = [rr[pl.ds(k * 16, 16)] for k in range(NK)]
            for q in range(Q):
                mr = mem_v.at[tl, q]
                acc = r[0] * mr[pl.ds(0, 16)]
                for k in range(1, NK):
                    acc = acc + r[k] * mr[pl.ds(k * 16, 16)]
                s = jnp.sum(acc)
                sl = tl * Q + q
                cu = minv_s[sl]
                bu = minb_s[sl]
                upd = s < cu
                minv_s[sl] = jnp.where(upd, s, cu)
                minb_s[sl] = jnp.where(upd, bg, bu)
            return c2
        lax.fori_loop(0, nj, _b, 0)
        return carry
    nchunks = (nb + CHUNK - 1) // CHUNK
    lax.fori_loop(0, nchunks, _chunk, 0)

    zi = jnp.zeros((16,), jnp.int32)
    for g in range(RPW // 16):
        vec = zi
        for l in range(16):
            vec = jnp.where(lane_iota == l,
                            jnp.maximum(minb_s[g * 16 + l], 0), vec)
        idx_v[pl.ds(g * 16, 16)] = vec
    pltpu.async_copy(reprs_hbm.at[idx_v], chos_v, sem).wait()
    pltpu.sync_copy(chos_v, chosen_hbm.at[pl.ds(lo * Q, RPW)])


_sc_call = pl.kernel(
    _sc_update,
    out_type=jax.ShapeDtypeStruct((N_TRACKS * Q, N), jnp.float32),
    mesh=plsc.VectorSubcoreMesh(core_axis_name="c", subcore_axis_name="s",
                                num_cores=NC, num_subcores=NS),
    compiler_params=pltpu.CompilerParams(needs_layout_passes=False),
    scratch_types=[
        pltpu.VMEM((B,), jnp.int32),
        pltpu.VMEM((B + CHUNK,), jnp.int32),
        pltpu.VMEM((NCH + 1, CHUNK), jnp.int32),
        pltpu.VMEM((B + CHUNK,), jnp.int32),
        pltpu.VMEM((CHUNK, N), jnp.float32),
        pltpu.VMEM((TPW, Q, N), jnp.float32),
        pltpu.VMEM((RPW, N), jnp.float32),
        pltpu.VMEM((RPW,), jnp.int32),
        pltpu.SMEM((RPW,), jnp.float32),
        pltpu.SMEM((RPW,), jnp.int32),
        pltpu.SemaphoreType.DMA,
    ],
)


def _finish_kernel(mem_ref, chosen_ref, tids_ref, alpha_ref, out_ref):
    mem = mem_ref[...]
    ch = chosen_ref[...].reshape(N_TRACKS, Q, N)
    a = alpha_ref[...].reshape(1, Q, N)
    new = mem * a + ch * (1.0 - a)
    nrm = jnp.sqrt(jnp.sum(new * new, axis=-1, keepdims=True))
    new = new / (nrm + EPS)
    row_t = jax.lax.broadcasted_iota(jnp.int32, (N_TRACKS, B), 0)
    present = jnp.any(row_t == tids_ref[...].reshape(1, B), axis=1)
    out_ref[...] = jnp.where(present.reshape(N_TRACKS, 1, 1), new, mem)


@jax.jit
def kernel(reprs, track_idxs, memory, alpha):
    tids = track_idxs.astype(jnp.int32)
    chosen = _sc_call(reprs, tids, memory)
    alpha_b = jnp.broadcast_to(alpha.reshape(Q, 1), (Q, N))
    out = pl.pallas_call(
        _finish_kernel,
        out_shape=jax.ShapeDtypeStruct((N_TRACKS, Q, N), jnp.float32),
    )(memory, chosen, tids.reshape(1, B), alpha_b)
    return out
